# baseline (speedup 1.0000x reference)
import functools

import jax
import jax.numpy as jnp
from jax import lax
from jax.experimental import pallas as pl
from jax.experimental.pallas import tpu as pltpu
from jax.experimental.pallas import tpu_sc as plsc

DIM = 64
L = 50
EX_PER_CHUNK = 2
CHUNK = EX_PER_CHUNK * L
NC, NS = 2, 16
NW = NC * NS
NVREG = DIM // 16
ROW = 2 * DIM
WIN = 128
NB = DIM // 8

_SC_PARAMS = pltpu.CompilerParams(needs_layout_passes=False)


def _make_repack(vocab: int):
    full = vocab // WIN
    tail = vocab - full * WIN

    mesh = plsc.VectorSubcoreMesh(core_axis_name="c", subcore_axis_name="s")

    @functools.partial(
        pl.kernel,
        out_type=jax.ShapeDtypeStruct((vocab // 2, ROW), jnp.float32),
        mesh=mesh,
        scratch_types=[
            pltpu.VMEM((NB, 8, WIN), jnp.float32),
            pltpu.VMEM((NB, 8, WIN), jnp.float32),
            pltpu.VMEM((WIN // 2, ROW), jnp.float32),
            pltpu.VMEM((WIN // 2, ROW), jnp.float32),
            pltpu.SemaphoreType.DMA,
            pltpu.SemaphoreType.DMA,
            pltpu.SemaphoreType.DMA,
            pltpu.SemaphoreType.DMA,
        ],
        compiler_params=_SC_PARAMS,
    )
    def repack(embT_hbm, tail_hbm, out_hbm, in0, in1, st0, st1,
               si0, si1, so0, so1):
        wid = lax.axis_index("c") * NS + lax.axis_index("s")
        lanes = lax.iota(jnp.int32, 16)
        bhi = lanes >> 3
        slo = lanes & 7

        def start_in(c, in_v, sem):
            for b in range(NB):
                pltpu.async_copy(
                    embT_hbm.at[pl.ds(8 * b, 8), pl.ds(c * WIN, WIN)],
                    in_v.at[b], sem)

        def wait_in(c, in_v, sem):
            for b in range(NB):
                pltpu.make_async_copy(
                    embT_hbm.at[pl.ds(8 * b, 8), pl.ds(c * WIN, WIN)],
                    in_v.at[b], sem).wait()

        def transpose_win(in_v, st_v, nv):
            @pl.loop(0, nv)
            def _(v):
                col = jnp.full((16,), v, jnp.int32)
                row = v >> 1
                half = (v & 1) * DIM
                for cg in range(NVREG):
                    vals = plsc.load_gather(
                        in_v, [2 * cg + bhi, slo, col])
                    st_v[row, pl.ds(half + cg * 16, 16)] = vals

        def start_out(c, st_v, sem):
            pltpu.async_copy(
                st_v, out_hbm.at[pl.ds(c * (WIN // 2), WIN // 2)], sem)

        def wait_out(c, st_v, sem):
            pltpu.make_async_copy(
                st_v, out_hbm.at[pl.ds(c * (WIN // 2), WIN // 2)], sem).wait()

        max_trips = (full + NW - 1) // NW

        @pl.when(wid < full)
        def _():
            start_in(wid, in0, si0)

        @pl.when(wid + NW < full)
        def _():
            start_in(wid + NW, in1, si1)

        @pl.loop(0, max_trips, step=2)
        def _(k):
            c0 = wid + k * NW
            c1 = wid + (k + 1) * NW

            @pl.when(c0 < full)
            def _():
                wait_in(c0, in0, si0)

                @pl.when(k >= 2)
                def _():
                    wait_out(c0 - 2 * NW, st0, so0)
                transpose_win(in0, st0, WIN)

                @pl.when(c0 + 2 * NW < full)
                def _():
                    start_in(c0 + 2 * NW, in0, si0)
                start_out(c0, st0, so0)

            @pl.when(c1 < full)
            def _():
                wait_in(c1, in1, si1)

                @pl.when(k >= 2)
                def _():
                    wait_out(c1 - 2 * NW, st1, so1)
                transpose_win(in1, st1, WIN)

                @pl.when(c1 + 2 * NW < full)
                def _():
                    start_in(c1 + 2 * NW, in1, si1)
                start_out(c1, st1, so1)

        n_mine = (full - wid + NW - 1) // NW
        last = wid + (n_mine - 1) * NW

        @pl.when(n_mine >= 1)
        def _():
            @pl.when(((n_mine - 1) % 2) == 0)
            def _():
                wait_out(last, st0, so0)

            @pl.when(((n_mine - 1) % 2) == 1)
            def _():
                wait_out(last, st1, so1)

        @pl.when(n_mine >= 2)
        def _():
            @pl.when(((n_mine - 2) % 2) == 0)
            def _():
                wait_out(last - NW, st0, so0)

            @pl.when(((n_mine - 2) % 2) == 1)
            def _():
                wait_out(last - NW, st1, so1)

        if tail:
            @pl.when(wid == 0)
            def _():
                for b in range(NB):
                    pltpu.sync_copy(
                        tail_hbm.at[pl.ds(8 * b, 8)], in0.at[b])
                transpose_win(in0, st0, tail)
                pltpu.sync_copy(
                    st0.at[pl.ds(0, tail // 2)],
                    out_hbm.at[pl.ds(full * (WIN // 2), tail // 2)])

    return repack


def _make_pool(total_chunks: int):
    tpw = total_chunks // NW
    epw = tpw * EX_PER_CHUNK

    mesh = plsc.VectorSubcoreMesh(core_axis_name="c", subcore_axis_name="s")

    @functools.partial(
        pl.kernel,
        out_type=jax.ShapeDtypeStruct((NW, epw * DIM), jnp.float32),
        mesh=mesh,
        scratch_types=[
            pltpu.VMEM((tpw, CHUNK), jnp.int32),
            pltpu.VMEM((tpw, CHUNK), jnp.int32),
            pltpu.VMEM((CHUNK, ROW), jnp.float32),
            pltpu.VMEM((CHUNK, ROW), jnp.float32),
            pltpu.VMEM((epw * DIM,), jnp.float32),
            pltpu.SemaphoreType.DMA,
            pltpu.SemaphoreType.DMA,
        ],
        compiler_params=_SC_PARAMS,
    )
    def pool(idx_hbm, par_hbm, emb_hbm, out_hbm, idx_v, par_v, rows0, rows1,
             out_v, sem0, sem1):
        wid = lax.axis_index("c") * NS + lax.axis_index("s")
        pltpu.sync_copy(idx_hbm.at[pl.ds(wid * tpw, tpw)], idx_v)
        pltpu.sync_copy(par_hbm.at[pl.ds(wid * tpw, tpw)], par_v)
        lanes = lax.iota(jnp.int32, 16)

        def reduce_chunk(rows, i, local_base):
            for e in range(EX_PER_CHUNK):
                acc = [jnp.zeros((16,), jnp.float32) for _ in range(NVREG)]
                for r in range(L):
                    g = min((r // 16) * 16, L - 16)
                    if r % 16 == 0:
                        par_vec = par_v[i, pl.ds(e * L + g, 16)]
                    coloff = DIM * jnp.sum(
                        jnp.where(lanes == (r - g), par_vec, 0))
                    for c in range(NVREG):
                        acc[c] = acc[c] + rows[e * L + r,
                                               pl.ds(coloff + c * 16, 16)]
                off = (local_base + e) * DIM
                for c in range(NVREG):
                    out_v[pl.ds(off + c * 16, 16)] = acc[c]

        pltpu.async_copy(emb_hbm.at[idx_v.at[0]], rows0, sem0)
        pltpu.async_copy(emb_hbm.at[idx_v.at[1]], rows1, sem1)

        @pl.loop(0, tpw, step=2)
        def _(i):
            pltpu.make_async_copy(emb_hbm.at[idx_v.at[i]], rows0, sem0).wait()
            reduce_chunk(rows0, i, EX_PER_CHUNK * i)

            @pl.when(i + 2 < tpw)
            def _():
                pltpu.async_copy(emb_hbm.at[idx_v.at[i + 2]], rows0, sem0)

            pltpu.make_async_copy(emb_hbm.at[idx_v.at[i + 1]], rows1, sem1).wait()
            reduce_chunk(rows1, i + 1, EX_PER_CHUNK * (i + 1))

            @pl.when(i + 3 < tpw)
            def _():
                pltpu.async_copy(emb_hbm.at[idx_v.at[i + 3]], rows1, sem1)

        pltpu.sync_copy(out_v, out_hbm.at[wid])

    return pool


def _linear(e0, e1, l0, l1, w0, w1, bias):
    B = e0.shape[0]
    labels = w0.shape[0]
    bm = 512

    def body(e0_ref, e1_ref, l0_ref, l1_ref, w0_ref, w1_ref, b_ref, out_ref):
        s0 = e0_ref[...] / l0_ref[...]
        s1 = e1_ref[...] / l1_ref[...]
        dn = (((1,), (1,)), ((), ()))
        p = lax.dot_general(w0_ref[...], s0, dn, preferred_element_type=jnp.float32)
        p = p + lax.dot_general(w1_ref[...], s1, dn, preferred_element_type=jnp.float32)
        out_ref[...] = p + b_ref[...]

    return pl.pallas_call(
        body,
        grid=(B // bm,),
        in_specs=[
            pl.BlockSpec((bm, DIM), lambda m: (m, 0)),
            pl.BlockSpec((bm, DIM), lambda m: (m, 0)),
            pl.BlockSpec((bm, 1), lambda m: (m, 0)),
            pl.BlockSpec((bm, 1), lambda m: (m, 0)),
            pl.BlockSpec((labels, DIM), lambda m: (0, 0)),
            pl.BlockSpec((labels, DIM), lambda m: (0, 0)),
            pl.BlockSpec((labels, 1), lambda m: (0, 0)),
        ],
        out_specs=pl.BlockSpec((labels, bm), lambda m: (0, m)),
        out_shape=jax.ShapeDtypeStruct((labels, B), jnp.float32),
    )(e0, e1, l0, l1, w0, w1, bias)


def kernel(x0, x1, length_0, length_1, embedding, W, b):
    B, seq = x0.shape
    vocab = embedding.shape[0]
    assert seq == L and embedding.shape[1] == DIM
    x = jnp.concatenate([x0, x1], axis=0).astype(jnp.int32)
    idx2 = (x >> 1).reshape(-1, CHUNK)
    par2 = (x & 1).reshape(-1, CHUNK)
    total_chunks = idx2.shape[0]

    tail_rows = vocab - (vocab // WIN) * WIN
    tail = jnp.pad(embedding[vocab - tail_rows:].T,
                   ((0, 0), (0, WIN - tail_rows)))
    emb2 = _make_repack(vocab)(embedding.T, tail)
    pooled = _make_pool(total_chunks)(idx2, par2, emb2)
    half = NW // 2
    e0 = pooled[:half].reshape(B, DIM)
    e1 = pooled[half:].reshape(B, DIM)

    pred_t = _linear(
        e0, e1,
        length_0.reshape(B, 1), length_1.reshape(B, 1),
        W[:, :DIM], W[:, DIM:],
        b.reshape(-1, 1),
    )
    return (pred_t.T, 0.0)


# speedup vs baseline: 2.4322x; 2.4322x over previous
import functools

import jax
import jax.numpy as jnp
from jax import lax
from jax.experimental import pallas as pl
from jax.experimental.pallas import tpu as pltpu
from jax.experimental.pallas import tpu_sc as plsc

DIM = 64
L = 50
EX_PER_CHUNK = 2
CHUNK = EX_PER_CHUNK * L
NC, NS = 2, 16
NW = NC * NS
NVREG = DIM // 16
ROW = 128


def _make_pool(total_chunks: int):
    tpw = total_chunks // NW
    epw = tpw * EX_PER_CHUNK

    mesh = plsc.VectorSubcoreMesh(core_axis_name="c", subcore_axis_name="s")

    @functools.partial(
        pl.kernel,
        out_type=jax.ShapeDtypeStruct((NW, epw * DIM), jnp.float32),
        mesh=mesh,
        scratch_types=[
            pltpu.VMEM((tpw, CHUNK), jnp.int32),
            pltpu.VMEM((CHUNK, ROW), jnp.float32),
            pltpu.VMEM((CHUNK, ROW), jnp.float32),
            pltpu.VMEM((epw * DIM,), jnp.float32),
            pltpu.SemaphoreType.DMA,
            pltpu.SemaphoreType.DMA,
        ],
    )
    def pool(idx_hbm, emb_hbm, out_hbm, idx_v, rows0, rows1, out_v, sem0, sem1):
        wid = lax.axis_index("c") * NS + lax.axis_index("s")
        pltpu.sync_copy(idx_hbm.at[pl.ds(wid * tpw, tpw)], idx_v)

        def reduce_chunk(rows, local_base):
            for e in range(EX_PER_CHUNK):
                acc = [rows[e * L, pl.ds(c * 16, 16)] for c in range(NVREG)]
                for r in range(1, L):
                    for c in range(NVREG):
                        acc[c] = acc[c] + rows[e * L + r, pl.ds(c * 16, 16)]
                off = (local_base + e) * DIM
                for c in range(NVREG):
                    out_v[pl.ds(off + c * 16, 16)] = acc[c]

        pltpu.async_copy(emb_hbm.at[idx_v.at[0]], rows0, sem0)
        pltpu.async_copy(emb_hbm.at[idx_v.at[1]], rows1, sem1)

        @pl.loop(0, tpw, step=2)
        def _(i):
            pltpu.make_async_copy(emb_hbm.at[idx_v.at[i]], rows0, sem0).wait()
            reduce_chunk(rows0, EX_PER_CHUNK * i)

            @pl.when(i + 2 < tpw)
            def _():
                pltpu.async_copy(emb_hbm.at[idx_v.at[i + 2]], rows0, sem0)

            pltpu.make_async_copy(emb_hbm.at[idx_v.at[i + 1]], rows1, sem1).wait()
            reduce_chunk(rows1, EX_PER_CHUNK * (i + 1))

            @pl.when(i + 3 < tpw)
            def _():
                pltpu.async_copy(emb_hbm.at[idx_v.at[i + 3]], rows1, sem1)

        pltpu.sync_copy(out_v, out_hbm.at[wid])

    return pool


def _linear(e0, e1, l0, l1, w0, w1, bias):
    B = e0.shape[0]
    labels = w0.shape[0]
    bm = 512

    def body(e0_ref, e1_ref, l0_ref, l1_ref, w0_ref, w1_ref, b_ref, out_ref):
        s0 = e0_ref[...] / l0_ref[...]
        s1 = e1_ref[...] / l1_ref[...]
        dn = (((1,), (1,)), ((), ()))
        p = lax.dot_general(w0_ref[...], s0, dn, preferred_element_type=jnp.float32)
        p = p + lax.dot_general(w1_ref[...], s1, dn, preferred_element_type=jnp.float32)
        out_ref[...] = p + b_ref[...]

    return pl.pallas_call(
        body,
        grid=(B // bm,),
        in_specs=[
            pl.BlockSpec((bm, DIM), lambda m: (m, 0)),
            pl.BlockSpec((bm, DIM), lambda m: (m, 0)),
            pl.BlockSpec((bm, 1), lambda m: (m, 0)),
            pl.BlockSpec((bm, 1), lambda m: (m, 0)),
            pl.BlockSpec((labels, DIM), lambda m: (0, 0)),
            pl.BlockSpec((labels, DIM), lambda m: (0, 0)),
            pl.BlockSpec((labels, 1), lambda m: (0, 0)),
        ],
        out_specs=pl.BlockSpec((labels, bm), lambda m: (0, m)),
        out_shape=jax.ShapeDtypeStruct((labels, B), jnp.float32),
    )(e0, e1, l0, l1, w0, w1, bias)


def kernel(x0, x1, length_0, length_1, embedding, W, b):
    B, seq = x0.shape
    assert seq == L and embedding.shape[1] == DIM
    x = jnp.concatenate([x0, x1], axis=0).astype(jnp.int32)
    idx2 = x.reshape(-1, CHUNK)
    total_chunks = idx2.shape[0]

    embp = jnp.pad(embedding, ((0, 0), (0, ROW - DIM)))
    pooled = _make_pool(total_chunks)(idx2, embp)
    half = NW // 2
    e0 = pooled[:half].reshape(B, DIM)
    e1 = pooled[half:].reshape(B, DIM)

    pred_t = _linear(
        e0, e1,
        length_0.reshape(B, 1), length_1.reshape(B, 1),
        W[:, :DIM], W[:, DIM:],
        b.reshape(-1, 1),
    )
    return (pred_t.T, 0.0)


# speedup vs baseline: 3.3110x; 1.3613x over previous
import functools

import jax
import jax.numpy as jnp
from jax import lax
from jax.experimental import pallas as pl
from jax.experimental.pallas import tpu as pltpu
from jax.experimental.pallas import tpu_sc as plsc

DIM = 64
L = 50
EX_PER_CHUNK = 2
CHUNK = EX_PER_CHUNK * L
NC, NS = 2, 16
NW = NC * NS
NVREG = DIM // 16
ROW = 128


def _repack_table(embT, eye):
    V = embT.shape[1]
    bw = 4096

    def body(in_ref, eye_ref, out_ref):
        t = lax.dot_general(in_ref[...], eye_ref[...],
                            (((0,), (0,)), ((), ())),
                            preferred_element_type=jnp.float32)
        out_ref[...] = jnp.concatenate([t, jnp.zeros_like(t)], axis=1)

    return pl.pallas_call(
        body,
        grid=(pl.cdiv(V, bw),),
        in_specs=[
            pl.BlockSpec((DIM, bw), lambda m: (0, m)),
            pl.BlockSpec((DIM, DIM), lambda m: (0, 0)),
        ],
        out_specs=pl.BlockSpec((bw, ROW), lambda m: (m, 0)),
        out_shape=jax.ShapeDtypeStruct((V, ROW), jnp.float32),
    )(embT, eye)


def _make_pool(total_chunks: int):
    tpw = total_chunks // NW
    epw = tpw * EX_PER_CHUNK

    mesh = plsc.VectorSubcoreMesh(core_axis_name="c", subcore_axis_name="s")

    @functools.partial(
        pl.kernel,
        out_type=jax.ShapeDtypeStruct((NW, epw * DIM), jnp.float32),
        mesh=mesh,
        scratch_types=[
            pltpu.VMEM((tpw, CHUNK), jnp.int32),
            pltpu.VMEM((CHUNK, ROW), jnp.float32),
            pltpu.VMEM((CHUNK, ROW), jnp.float32),
            pltpu.VMEM((epw * DIM,), jnp.float32),
            pltpu.SemaphoreType.DMA,
            pltpu.SemaphoreType.DMA,
        ],
    )
    def pool(idx_hbm, emb_hbm, out_hbm, idx_v, rows0, rows1, out_v, sem0, sem1):
        wid = lax.axis_index("c") * NS + lax.axis_index("s")
        pltpu.sync_copy(idx_hbm.at[pl.ds(wid * tpw, tpw)], idx_v)

        def reduce_chunk(rows, local_base):
            for e in range(EX_PER_CHUNK):
                acc = [rows[e * L, pl.ds(c * 16, 16)] for c in range(NVREG)]
                for r in range(1, L):
                    for c in range(NVREG):
                        acc[c] = acc[c] + rows[e * L + r, pl.ds(c * 16, 16)]
                off = (local_base + e) * DIM
                for c in range(NVREG):
                    out_v[pl.ds(off + c * 16, 16)] = acc[c]

        pltpu.async_copy(emb_hbm.at[idx_v.at[0]], rows0, sem0)
        pltpu.async_copy(emb_hbm.at[idx_v.at[1]], rows1, sem1)

        @pl.loop(0, tpw, step=2)
        def _(i):
            pltpu.make_async_copy(emb_hbm.at[idx_v.at[i]], rows0, sem0).wait()
            reduce_chunk(rows0, EX_PER_CHUNK * i)

            @pl.when(i + 2 < tpw)
            def _():
                pltpu.async_copy(emb_hbm.at[idx_v.at[i + 2]], rows0, sem0)

            pltpu.make_async_copy(emb_hbm.at[idx_v.at[i + 1]], rows1, sem1).wait()
            reduce_chunk(rows1, EX_PER_CHUNK * (i + 1))

            @pl.when(i + 3 < tpw)
            def _():
                pltpu.async_copy(emb_hbm.at[idx_v.at[i + 3]], rows1, sem1)

        pltpu.sync_copy(out_v, out_hbm.at[wid])

    return pool


def _linear(e0, e1, l0, l1, w0, w1, bias):
    B = e0.shape[0]
    labels = w0.shape[0]
    bm = 512

    def body(e0_ref, e1_ref, l0_ref, l1_ref, w0_ref, w1_ref, b_ref, out_ref):
        s0 = e0_ref[...] / l0_ref[...]
        s1 = e1_ref[...] / l1_ref[...]
        dn = (((1,), (1,)), ((), ()))
        p = lax.dot_general(w0_ref[...], s0, dn, preferred_element_type=jnp.float32)
        p = p + lax.dot_general(w1_ref[...], s1, dn, preferred_element_type=jnp.float32)
        out_ref[...] = p + b_ref[...]

    return pl.pallas_call(
        body,
        grid=(B // bm,),
        in_specs=[
            pl.BlockSpec((bm, DIM), lambda m: (m, 0)),
            pl.BlockSpec((bm, DIM), lambda m: (m, 0)),
            pl.BlockSpec((bm, 1), lambda m: (m, 0)),
            pl.BlockSpec((bm, 1), lambda m: (m, 0)),
            pl.BlockSpec((labels, DIM), lambda m: (0, 0)),
            pl.BlockSpec((labels, DIM), lambda m: (0, 0)),
            pl.BlockSpec((labels, 1), lambda m: (0, 0)),
        ],
        out_specs=pl.BlockSpec((labels, bm), lambda m: (0, m)),
        out_shape=jax.ShapeDtypeStruct((labels, B), jnp.float32),
    )(e0, e1, l0, l1, w0, w1, bias)


def kernel(x0, x1, length_0, length_1, embedding, W, b):
    B, seq = x0.shape
    assert seq == L and embedding.shape[1] == DIM
    x = jnp.concatenate([x0, x1], axis=0).astype(jnp.int32)
    idx2 = x.reshape(-1, CHUNK)
    total_chunks = idx2.shape[0]

    embp = _repack_table(embedding.T, jnp.eye(DIM, dtype=jnp.float32))
    pooled = _make_pool(total_chunks)(idx2, embp)
    half = NW // 2
    e0 = pooled[:half].reshape(B, DIM)
    e1 = pooled[half:].reshape(B, DIM)

    pred_t = _linear(
        e0, e1,
        length_0.reshape(B, 1), length_1.reshape(B, 1),
        W[:, :DIM], W[:, DIM:],
        b.reshape(-1, 1),
    )
    return (pred_t.T, 0.0)


# speedup vs baseline: 3.3620x; 1.0154x over previous
import functools

import jax
import jax.numpy as jnp
from jax import lax
from jax.experimental import pallas as pl
from jax.experimental.pallas import tpu as pltpu
from jax.experimental.pallas import tpu_sc as plsc

DIM = 64
L = 50
EX_PER_CHUNK = 2
CHUNK = EX_PER_CHUNK * L
NC, NS = 2, 16
NW = NC * NS
NVREG = DIM // 16
ROW = 128


def _repack_table(embT, eye):
    V = embT.shape[1]
    bw = 4096

    def body(in_ref, eye_ref, out_ref):
        del eye_ref
        t = jnp.transpose(in_ref[...])
        out_ref[...] = jnp.concatenate([t, jnp.zeros_like(t)], axis=1)

    return pl.pallas_call(
        body,
        grid=(pl.cdiv(V, bw),),
        in_specs=[
            pl.BlockSpec((DIM, bw), lambda m: (0, m)),
            pl.BlockSpec((DIM, DIM), lambda m: (0, 0)),
        ],
        out_specs=pl.BlockSpec((bw, ROW), lambda m: (m, 0)),
        out_shape=jax.ShapeDtypeStruct((V, ROW), jnp.float32),
    )(embT, eye)


def _make_pool(total_chunks: int):
    tpw = total_chunks // NW
    epw = tpw * EX_PER_CHUNK

    mesh = plsc.VectorSubcoreMesh(core_axis_name="c", subcore_axis_name="s")

    @functools.partial(
        pl.kernel,
        out_type=jax.ShapeDtypeStruct((NW, epw * DIM), jnp.float32),
        mesh=mesh,
        scratch_types=[
            pltpu.VMEM((tpw, CHUNK), jnp.int32),
            pltpu.VMEM((CHUNK, ROW), jnp.float32),
            pltpu.VMEM((CHUNK, ROW), jnp.float32),
            pltpu.VMEM((epw * DIM,), jnp.float32),
            pltpu.SemaphoreType.DMA,
            pltpu.SemaphoreType.DMA,
        ],
    )
    def pool(idx_hbm, emb_hbm, out_hbm, idx_v, rows0, rows1, out_v, sem0, sem1):
        wid = lax.axis_index("c") * NS + lax.axis_index("s")
        pltpu.sync_copy(idx_hbm.at[pl.ds(wid * tpw, tpw)], idx_v)

        def reduce_chunk(rows, local_base):
            for e in range(EX_PER_CHUNK):
                acc = [rows[e * L, pl.ds(c * 16, 16)] for c in range(NVREG)]
                for r in range(1, L):
                    for c in range(NVREG):
                        acc[c] = acc[c] + rows[e * L + r, pl.ds(c * 16, 16)]
                off = (local_base + e) * DIM
                for c in range(NVREG):
                    out_v[pl.ds(off + c * 16, 16)] = acc[c]

        pltpu.async_copy(emb_hbm.at[idx_v.at[0]], rows0, sem0)
        pltpu.async_copy(emb_hbm.at[idx_v.at[1]], rows1, sem1)

        @pl.loop(0, tpw, step=2)
        def _(i):
            pltpu.make_async_copy(emb_hbm.at[idx_v.at[i]], rows0, sem0).wait()
            reduce_chunk(rows0, EX_PER_CHUNK * i)

            @pl.when(i + 2 < tpw)
            def _():
                pltpu.async_copy(emb_hbm.at[idx_v.at[i + 2]], rows0, sem0)

            pltpu.make_async_copy(emb_hbm.at[idx_v.at[i + 1]], rows1, sem1).wait()
            reduce_chunk(rows1, EX_PER_CHUNK * (i + 1))

            @pl.when(i + 3 < tpw)
            def _():
                pltpu.async_copy(emb_hbm.at[idx_v.at[i + 3]], rows1, sem1)

        pltpu.sync_copy(out_v, out_hbm.at[wid])

    return pool


def _linear(e0, e1, l0, l1, w0, w1, bias):
    B = e0.shape[0]
    labels = w0.shape[0]
    bm = 512

    def body(e0_ref, e1_ref, l0_ref, l1_ref, w0_ref, w1_ref, b_ref, out_ref):
        s0 = e0_ref[...] / l0_ref[...]
        s1 = e1_ref[...] / l1_ref[...]
        dn = (((1,), (1,)), ((), ()))
        p = lax.dot_general(w0_ref[...], s0, dn, preferred_element_type=jnp.float32)
        p = p + lax.dot_general(w1_ref[...], s1, dn, preferred_element_type=jnp.float32)
        out_ref[...] = p + b_ref[...]

    return pl.pallas_call(
        body,
        grid=(B // bm,),
        in_specs=[
            pl.BlockSpec((bm, DIM), lambda m: (m, 0)),
            pl.BlockSpec((bm, DIM), lambda m: (m, 0)),
            pl.BlockSpec((bm, 1), lambda m: (m, 0)),
            pl.BlockSpec((bm, 1), lambda m: (m, 0)),
            pl.BlockSpec((labels, DIM), lambda m: (0, 0)),
            pl.BlockSpec((labels, DIM), lambda m: (0, 0)),
            pl.BlockSpec((labels, 1), lambda m: (0, 0)),
        ],
        out_specs=pl.BlockSpec((labels, bm), lambda m: (0, m)),
        out_shape=jax.ShapeDtypeStruct((labels, B), jnp.float32),
    )(e0, e1, l0, l1, w0, w1, bias)


def kernel(x0, x1, length_0, length_1, embedding, W, b):
    B, seq = x0.shape
    assert seq == L and embedding.shape[1] == DIM
    x = jnp.concatenate([x0, x1], axis=0).astype(jnp.int32)
    idx2 = x.reshape(-1, CHUNK)
    total_chunks = idx2.shape[0]

    embp = _repack_table(embedding.T, jnp.eye(DIM, dtype=jnp.float32))
    pooled = _make_pool(total_chunks)(idx2, embp)
    half = NW // 2
    e0 = pooled[:half].reshape(B, DIM)
    e1 = pooled[half:].reshape(B, DIM)

    pred_t = _linear(
        e0, e1,
        length_0.reshape(B, 1), length_1.reshape(B, 1),
        W[:, :DIM], W[:, DIM:],
        b.reshape(-1, 1),
    )
    return (pred_t.T, 0.0)


# speedup vs baseline: 3.9387x; 1.1716x over previous
import functools

import jax
import jax.numpy as jnp
from jax import lax
from jax.experimental import pallas as pl
from jax.experimental.pallas import tpu as pltpu
from jax.experimental.pallas import tpu_sc as plsc

DIM = 64
L = 50
EX_PER_CHUNK = 2
CHUNK = EX_PER_CHUNK * L
NC, NS = 2, 16
NW = NC * NS
NVREG = DIM // 16
ROW = 128


def _repack_table(embT):
    V = embT.shape[1]
    bw = 8192

    def body(in_ref, out_ref):
        t = jnp.transpose(in_ref[...])
        out_ref[...] = jnp.concatenate([t, jnp.zeros_like(t)], axis=1)

    return pl.pallas_call(
        body,
        grid=(pl.cdiv(V, bw),),
        in_specs=[pl.BlockSpec((DIM, bw), lambda m: (0, m))],
        out_specs=pl.BlockSpec((bw, ROW), lambda m: (m, 0)),
        out_shape=jax.ShapeDtypeStruct((V, ROW), jnp.float32),
    )(embT)


def _make_pool(total_chunks: int):
    tpw = total_chunks // NW
    epw = tpw * EX_PER_CHUNK

    mesh = plsc.VectorSubcoreMesh(core_axis_name="c", subcore_axis_name="s")

    @functools.partial(
        pl.kernel,
        out_type=jax.ShapeDtypeStruct((NW, epw * DIM), jnp.float32),
        mesh=mesh,
        scratch_types=[
            pltpu.VMEM((tpw, CHUNK), jnp.int32),
            pltpu.VMEM((CHUNK, ROW), jnp.float32),
            pltpu.VMEM((CHUNK, ROW), jnp.float32),
            pltpu.VMEM((epw * DIM,), jnp.float32),
            pltpu.SemaphoreType.DMA,
            pltpu.SemaphoreType.DMA,
        ],
    )
    def pool(idx_hbm, emb_hbm, out_hbm, idx_v, rows0, rows1, out_v, sem0, sem1):
        wid = lax.axis_index("c") * NS + lax.axis_index("s")
        pltpu.sync_copy(idx_hbm.at[pl.ds(wid * tpw, tpw)], idx_v)

        def reduce_chunk(rows, local_base):
            for e in range(EX_PER_CHUNK):
                acc = [rows[e * L, pl.ds(c * 16, 16)] for c in range(NVREG)]
                for r in range(1, L):
                    for c in range(NVREG):
                        acc[c] = acc[c] + rows[e * L + r, pl.ds(c * 16, 16)]
                off = (local_base + e) * DIM
                for c in range(NVREG):
                    out_v[pl.ds(off + c * 16, 16)] = acc[c]

        pltpu.async_copy(emb_hbm.at[idx_v.at[0]], rows0, sem0)
        pltpu.async_copy(emb_hbm.at[idx_v.at[1]], rows1, sem1)

        @pl.loop(0, tpw, step=2)
        def _(i):
            pltpu.make_async_copy(emb_hbm.at[idx_v.at[i]], rows0, sem0).wait()
            reduce_chunk(rows0, EX_PER_CHUNK * i)

            @pl.when(i + 2 < tpw)
            def _():
                pltpu.async_copy(emb_hbm.at[idx_v.at[i + 2]], rows0, sem0)

            pltpu.make_async_copy(emb_hbm.at[idx_v.at[i + 1]], rows1, sem1).wait()
            reduce_chunk(rows1, EX_PER_CHUNK * (i + 1))

            @pl.when(i + 3 < tpw)
            def _():
                pltpu.async_copy(emb_hbm.at[idx_v.at[i + 3]], rows1, sem1)

        pltpu.sync_copy(out_v, out_hbm.at[wid])

    return pool


def _linear(e0, e1, l0, l1, w0, w1, bias):
    B = e0.shape[0]
    labels = w0.shape[0]
    bm = 512

    def body(e0_ref, e1_ref, l0_ref, l1_ref, w0_ref, w1_ref, b_ref, out_ref):
        s0 = e0_ref[...] / l0_ref[...]
        s1 = e1_ref[...] / l1_ref[...]
        dn = (((1,), (1,)), ((), ()))
        p = lax.dot_general(w0_ref[...], s0, dn, preferred_element_type=jnp.float32)
        p = p + lax.dot_general(w1_ref[...], s1, dn, preferred_element_type=jnp.float32)
        out_ref[...] = p + b_ref[...]

    return pl.pallas_call(
        body,
        grid=(B // bm,),
        in_specs=[
            pl.BlockSpec((bm, DIM), lambda m: (m, 0)),
            pl.BlockSpec((bm, DIM), lambda m: (m, 0)),
            pl.BlockSpec((bm, 1), lambda m: (m, 0)),
            pl.BlockSpec((bm, 1), lambda m: (m, 0)),
            pl.BlockSpec((labels, DIM), lambda m: (0, 0)),
            pl.BlockSpec((labels, DIM), lambda m: (0, 0)),
            pl.BlockSpec((labels, 1), lambda m: (0, 0)),
        ],
        out_specs=pl.BlockSpec((labels, bm), lambda m: (0, m)),
        out_shape=jax.ShapeDtypeStruct((labels, B), jnp.float32),
    )(e0, e1, l0, l1, w0, w1, bias)


def kernel(x0, x1, length_0, length_1, embedding, W, b):
    B, seq = x0.shape
    assert seq == L and embedding.shape[1] == DIM
    x = jnp.concatenate([x0, x1], axis=0).astype(jnp.int32)
    idx2 = x.reshape(-1, CHUNK)
    total_chunks = idx2.shape[0]

    embp = _repack_table(embedding.T)
    pooled = _make_pool(total_chunks)(idx2, embp)
    half = NW // 2
    e0 = pooled[:half].reshape(B, DIM)
    e1 = pooled[half:].reshape(B, DIM)

    pred_t = _linear(
        e0, e1,
        length_0.reshape(B, 1), length_1.reshape(B, 1),
        W[:, :DIM], W[:, DIM:],
        b.reshape(-1, 1),
    )
    return (pred_t.T, 0.0)


# speedup vs baseline: 4.1342x; 1.0496x over previous
import functools

import jax
import jax.numpy as jnp
from jax import lax
from jax.experimental import pallas as pl
from jax.experimental.pallas import tpu as pltpu
from jax.experimental.pallas import tpu_sc as plsc

DIM = 64
L = 50
EX_PER_CHUNK = 2
CHUNK = EX_PER_CHUNK * L
NC, NS = 2, 16
NW = NC * NS
NVREG = DIM // 16
ROW = 128


def _repack_table(embT):
    V = embT.shape[1]
    bw = 16384

    def body(in_ref, out_ref):
        t = jnp.transpose(in_ref[...])
        out_ref[...] = jnp.concatenate([t, jnp.zeros_like(t)], axis=1)

    return pl.pallas_call(
        body,
        grid=(pl.cdiv(V, bw),),
        in_specs=[pl.BlockSpec((DIM, bw), lambda m: (0, m))],
        out_specs=pl.BlockSpec((bw, ROW), lambda m: (m, 0)),
        out_shape=jax.ShapeDtypeStruct((V, ROW), jnp.float32),
    )(embT)


def _make_pool(total_chunks: int):
    tpw = total_chunks // NW
    epw = tpw * EX_PER_CHUNK

    mesh = plsc.VectorSubcoreMesh(core_axis_name="c", subcore_axis_name="s")

    @functools.partial(
        pl.kernel,
        out_type=jax.ShapeDtypeStruct((NW, epw * DIM), jnp.float32),
        mesh=mesh,
        scratch_types=[
            pltpu.VMEM((tpw, CHUNK), jnp.int32),
            pltpu.VMEM((CHUNK, ROW), jnp.float32),
            pltpu.VMEM((CHUNK, ROW), jnp.float32),
            pltpu.VMEM((epw * DIM,), jnp.float32),
            pltpu.SemaphoreType.DMA,
            pltpu.SemaphoreType.DMA,
        ],
    )
    def pool(idx_hbm, emb_hbm, out_hbm, idx_v, rows0, rows1, out_v, sem0, sem1):
        wid = lax.axis_index("c") * NS + lax.axis_index("s")
        pltpu.sync_copy(idx_hbm.at[pl.ds(wid * tpw, tpw)], idx_v)

        def reduce_chunk(rows, local_base):
            for e in range(EX_PER_CHUNK):
                acc = [rows[e * L, pl.ds(c * 16, 16)] for c in range(NVREG)]
                for r in range(1, L):
                    for c in range(NVREG):
                        acc[c] = acc[c] + rows[e * L + r, pl.ds(c * 16, 16)]
                off = (local_base + e) * DIM
                for c in range(NVREG):
                    out_v[pl.ds(off + c * 16, 16)] = acc[c]

        pltpu.async_copy(emb_hbm.at[idx_v.at[0]], rows0, sem0)
        pltpu.async_copy(emb_hbm.at[idx_v.at[1]], rows1, sem1)

        @pl.loop(0, tpw, step=2)
        def _(i):
            pltpu.make_async_copy(emb_hbm.at[idx_v.at[i]], rows0, sem0).wait()
            reduce_chunk(rows0, EX_PER_CHUNK * i)

            @pl.when(i + 2 < tpw)
            def _():
                pltpu.async_copy(emb_hbm.at[idx_v.at[i + 2]], rows0, sem0)

            pltpu.make_async_copy(emb_hbm.at[idx_v.at[i + 1]], rows1, sem1).wait()
            reduce_chunk(rows1, EX_PER_CHUNK * (i + 1))

            @pl.when(i + 3 < tpw)
            def _():
                pltpu.async_copy(emb_hbm.at[idx_v.at[i + 3]], rows1, sem1)

        pltpu.sync_copy(out_v, out_hbm.at[wid])

    return pool


def _linear(e0, e1, l0, l1, w0, w1, bias):
    B = e0.shape[0]
    labels = w0.shape[0]
    bm = 512

    def body(e0_ref, e1_ref, l0_ref, l1_ref, w0_ref, w1_ref, b_ref, out_ref):
        s0 = e0_ref[...] / l0_ref[...]
        s1 = e1_ref[...] / l1_ref[...]
        dn = (((1,), (1,)), ((), ()))
        p = lax.dot_general(w0_ref[...], s0, dn, preferred_element_type=jnp.float32)
        p = p + lax.dot_general(w1_ref[...], s1, dn, preferred_element_type=jnp.float32)
        out_ref[...] = p + b_ref[...]

    return pl.pallas_call(
        body,
        grid=(B // bm,),
        in_specs=[
            pl.BlockSpec((bm, DIM), lambda m: (m, 0)),
            pl.BlockSpec((bm, DIM), lambda m: (m, 0)),
            pl.BlockSpec((bm, 1), lambda m: (m, 0)),
            pl.BlockSpec((bm, 1), lambda m: (m, 0)),
            pl.BlockSpec((labels, DIM), lambda m: (0, 0)),
            pl.BlockSpec((labels, DIM), lambda m: (0, 0)),
            pl.BlockSpec((labels, 1), lambda m: (0, 0)),
        ],
        out_specs=pl.BlockSpec((labels, bm), lambda m: (0, m)),
        out_shape=jax.ShapeDtypeStruct((labels, B), jnp.float32),
    )(e0, e1, l0, l1, w0, w1, bias)


def kernel(x0, x1, length_0, length_1, embedding, W, b):
    B, seq = x0.shape
    assert seq == L and embedding.shape[1] == DIM
    x = jnp.concatenate([x0, x1], axis=0).astype(jnp.int32)
    idx2 = x.reshape(-1, CHUNK)
    total_chunks = idx2.shape[0]

    embp = _repack_table(embedding.T)
    pooled = _make_pool(total_chunks)(idx2, embp)
    half = NW // 2
    e0 = pooled[:half].reshape(B, DIM)
    e1 = pooled[half:].reshape(B, DIM)

    pred_t = _linear(
        e0, e1,
        length_0.reshape(B, 1), length_1.reshape(B, 1),
        W[:, :DIM], W[:, DIM:],
        b.reshape(-1, 1),
    )
    return (pred_t.T, 0.0)
